# baseline (speedup 1.0000x reference)
import functools

import jax
import jax.numpy as jnp
from jax import lax
from jax.experimental import pallas as pl
from jax.experimental.pallas import tpu as pltpu
from jax.experimental.pallas import tpu_sc as plsc

_VOCAB = 100000
_DIM = 64
_BATCH = 4096
_NC = 2
_NS = 16
_NW = _NC * _NS
_BPW = _BATCH // _NW


def _embed_gather(indices, table):
    mesh = plsc.VectorSubcoreMesh(core_axis_name="c", subcore_axis_name="s")

    @functools.partial(
        pl.kernel,
        mesh=mesh,
        compiler_params=pltpu.CompilerParams(use_tc_tiling_on_sc=False),
        out_type=jax.ShapeDtypeStruct((_BATCH, _DIM), jnp.float32),
        scratch_types=[
            pltpu.VMEM((_BPW,), jnp.int32),
            pltpu.VMEM((_BPW, _DIM), jnp.float32),
            pltpu.SemaphoreType.DMA,
        ],
    )
    def k(idx_hbm, table_hbm, out_hbm, idx_v, rows_v, sem):
        wid = lax.axis_index("s") * _NC + lax.axis_index("c")
        base = wid * _BPW
        pltpu.sync_copy(idx_hbm.at[pl.ds(base, _BPW)], idx_v)
        pltpu.async_copy(table_hbm.at[idx_v], rows_v, sem).wait()
        pltpu.sync_copy(rows_v, out_hbm.at[pl.ds(base, _BPW)])

    return k(indices, table)


def kernel(indices, table):
    return _embed_gather(indices.astype(jnp.int32), table)


# speedup vs baseline: 1.4344x; 1.4344x over previous
import functools

import jax
import jax.numpy as jnp
from jax import lax
from jax.experimental import pallas as pl
from jax.experimental.pallas import tpu as pltpu
from jax.experimental.pallas import tpu_sc as plsc

_VOCAB = 100000
_DIM = 64
_BATCH = 4096
_NC = 2
_NS = 16
_NW = _NC * _NS
_BPW = _BATCH // _NW


def _embed_gather(indices, table):
    mesh = plsc.VectorSubcoreMesh(core_axis_name="c", subcore_axis_name="s")

    @functools.partial(
        pl.kernel,
        mesh=mesh,
        out_type=jax.ShapeDtypeStruct((_BATCH, _DIM), jnp.float32),
        scratch_types=[
            pltpu.VMEM((_BPW,), jnp.int32),
            pltpu.VMEM((_BPW, _DIM), jnp.float32),
            pltpu.SemaphoreType.DMA,
        ],
    )
    def k(idx_hbm, table_hbm, out_hbm, idx_v, rows_v, sem):
        wid = lax.axis_index("s") * _NC + lax.axis_index("c")
        base = wid * _BPW
        pltpu.sync_copy(idx_hbm.at[pl.ds(base, _BPW)], idx_v)

        for c in range(_BPW // 16):
            iv = idx_v[pl.ds(c * 16, 16)]
            for j in range(16):
                pltpu.async_copy(
                    table_hbm.at[pl.ds(iv[j], 1)],
                    rows_v.at[pl.ds(c * 16 + j, 1)],
                    sem,
                )
        pltpu.make_async_copy(table_hbm.at[pl.ds(0, _BPW)], rows_v, sem).wait()
        pltpu.sync_copy(rows_v, out_hbm.at[pl.ds(base, _BPW)])

    return k(indices, table)


def kernel(indices, table):
    return _embed_gather(indices.astype(jnp.int32), table)


# speedup vs baseline: 2.5622x; 1.7862x over previous
import functools

import jax
import jax.numpy as jnp
from jax import lax
from jax.experimental import pallas as pl
from jax.experimental.pallas import tpu as pltpu
from jax.experimental.pallas import tpu_sc as plsc

_VOCAB = 100000
_DIM = 64
_BATCH = 4096
_NC = 2
_NS = 16
_NW = _NC * _NS
_ROWS_PER_W = _DIM // _NW


def _embed_gather_t(indices, table_t):
    mesh = plsc.VectorSubcoreMesh(core_axis_name="c", subcore_axis_name="s")

    @functools.partial(
        pl.kernel,
        mesh=mesh,
        compiler_params=pltpu.CompilerParams(needs_layout_passes=False),
        out_type=jax.ShapeDtypeStruct((_DIM, _BATCH), jnp.float32),
        scratch_types=[
            pltpu.VMEM((_BATCH,), jnp.int32),
            pltpu.VMEM((_VOCAB,), jnp.float32),
            pltpu.VMEM((_BATCH,), jnp.float32),
        ],
    )
    def k(idx_hbm, table_hbm, out_hbm, idx_v, row_v, outrow_v):
        wid = lax.axis_index("s") * _NC + lax.axis_index("c")
        pltpu.sync_copy(idx_hbm, idx_v)
        for p in range(_ROWS_PER_W):
            j = p * _NW + wid
            pltpu.sync_copy(table_hbm.at[j], row_v)

            def chunk(c, carry):
                base = c * 64
                for u in range(4):
                    iv = idx_v[pl.ds(base + u * 16, 16)]
                    outrow_v[pl.ds(base + u * 16, 16)] = plsc.load_gather(
                        row_v, [iv]
                    )
                return carry

            lax.fori_loop(0, _BATCH // 64, chunk, 0)
            pltpu.sync_copy(outrow_v, out_hbm.at[j])

    return k(indices, table_t)


def kernel(indices, table):
    out_t = _embed_gather_t(indices.astype(jnp.int32), table.T)
    return out_t.T
